# Initial kernel scaffold; baseline (speedup 1.0000x reference)
import jax
import jax.numpy as jnp
from jax.experimental import pallas as pl


def kernel(X, L, H, C, Wx, bx, Wh, bh, wc, bg):
    raise NotImplementedError("write your pallas kernel here")


# speedup vs baseline: 1.9819x; 1.9819x over previous
import jax
import jax.numpy as jnp
from jax.experimental import pallas as pl
from jax.experimental.pallas import tpu as pltpu

T_STEPS = 12
K = 3
N = 1024
C_IN = 32
HID = 64
B = 16
CAT = C_IN + HID
GOUT = 4 * HID


def _step_kernel(x_ref, L_ref, Wx_ref, Wh_ref, bx_ref, bh_ref, bg_ref,
                 wc_ref, h0_ref, c0_ref, out_ref, h_s, c_s):
    t = pl.program_id(0)
    b = pl.program_id(1)

    @pl.when(t == 0)
    def _init():
        h_s[b] = h0_ref[0]
        c_s[b] = c0_ref[0]

    Lm = L_ref[...]

    x_b = x_ref[0, 0]
    h_b = h_s[b]
    c_b = c_s[b]

    U0 = jnp.concatenate([x_b, h_b], axis=1)
    U1 = jnp.dot(U0.T, Lm.T, preferred_element_type=jnp.float32).T
    U2 = 2.0 * jnp.dot(U1.T, Lm.T, preferred_element_type=jnp.float32).T - U0

    xw = (jnp.dot(U0[:, :C_IN], Wx_ref[0, 0], preferred_element_type=jnp.float32)
          + jnp.dot(U1[:, :C_IN], Wx_ref[0, 1], preferred_element_type=jnp.float32))
    xw = (xw + jnp.dot(U2[:, :C_IN], Wx_ref[0, 2], preferred_element_type=jnp.float32)) + bx_ref[0, 0]
    hw = (jnp.dot(U0[:, C_IN:], Wh_ref[0, 0], preferred_element_type=jnp.float32)
          + jnp.dot(U1[:, C_IN:], Wh_ref[0, 1], preferred_element_type=jnp.float32))
    hw = (hw + jnp.dot(U2[:, C_IN:], Wh_ref[0, 2], preferred_element_type=jnp.float32)) + bh_ref[0, 0]
    pre = xw + hw

    wc_i = wc_ref[0, 0]
    wc_f = wc_ref[0, 1]
    wc_o = wc_ref[0, 2]
    bg_i = bg_ref[0, 0, :HID]
    bg_f = bg_ref[0, 0, HID:2 * HID]
    bg_c = bg_ref[0, 0, 2 * HID:3 * HID]
    bg_o = bg_ref[0, 0, 3 * HID:]

    Ig = jax.nn.sigmoid((pre[:, :HID] + wc_i * c_b) + bg_i)
    Fg = jax.nn.sigmoid((pre[:, HID:2 * HID] + wc_f * c_b) + bg_f)
    Tc = jnp.tanh(pre[:, 2 * HID:3 * HID] + bg_c)
    c_new = Fg * c_b + Ig * Tc
    Og = jax.nn.sigmoid((pre[:, 3 * HID:] + wc_o * c_new) + bg_o)
    h_new = Og * jnp.tanh(c_new)

    c_s[b] = c_new
    h_s[b] = h_new
    out_ref[0, 0] = h_new


def kernel(X, L, H, C, Wx, bx, Wh, bh, wc, bg):
    WxS = jnp.transpose(Wx, (0, 2, 3, 1, 4)).reshape(T_STEPS, K, C_IN, GOUT)
    WhS = jnp.transpose(Wh, (0, 2, 3, 1, 4)).reshape(T_STEPS, K, HID, GOUT)
    bxS = bx.reshape(T_STEPS, 1, GOUT)
    bhS = bh.reshape(T_STEPS, 1, GOUT)
    bgS = bg[:, :, 0].reshape(T_STEPS, 1, GOUT)
    wcr = wc[:, :, 0]

    h0 = H[0]
    c0 = C[0]

    out = pl.pallas_call(
        _step_kernel,
        grid=(T_STEPS, B),
        in_specs=[
            pl.BlockSpec((1, 1, N, C_IN), lambda t, b: (b, t, 0, 0)),
            pl.BlockSpec((N, N), lambda t, b: (0, 0)),
            pl.BlockSpec((1, K, C_IN, GOUT), lambda t, b: (t, 0, 0, 0)),
            pl.BlockSpec((1, K, HID, GOUT), lambda t, b: (t, 0, 0, 0)),
            pl.BlockSpec((1, 1, GOUT), lambda t, b: (t, 0, 0)),
            pl.BlockSpec((1, 1, GOUT), lambda t, b: (t, 0, 0)),
            pl.BlockSpec((1, 1, GOUT), lambda t, b: (t, 0, 0)),
            pl.BlockSpec((1, 3, HID), lambda t, b: (t, 0, 0)),
            pl.BlockSpec((1, N, HID), lambda t, b: (b, 0, 0)),
            pl.BlockSpec((1, N, HID), lambda t, b: (b, 0, 0)),
        ],
        out_specs=pl.BlockSpec((1, 1, N, HID), lambda t, b: (b, t, 0, 0)),
        out_shape=jax.ShapeDtypeStruct((B, T_STEPS, N, HID), jnp.float32),
        scratch_shapes=[
            pltpu.VMEM((B, N, HID), jnp.float32),
            pltpu.VMEM((B, N, HID), jnp.float32),
        ],
        compiler_params=pltpu.CompilerParams(
            dimension_semantics=("arbitrary", "arbitrary"),
        ),
    )(X, L, WxS, WhS, bxS, bhS, bgS, wcr, h0, c0)
    return out


# speedup vs baseline: 4.6304x; 2.3364x over previous
import jax
import jax.numpy as jnp
from jax.experimental import pallas as pl
from jax.experimental.pallas import tpu as pltpu

T_STEPS = 12
K = 3
N = 1024
C_IN = 32
HID = 64
B = 16
CAT = C_IN + HID
GOUT = 4 * HID


def _step_kernel(xT_ref, LT_ref, Wx_ref, Wh_ref, bx_ref, bh_ref, bg_ref,
                 wc_ref, one_ref, h0_ref, c0_ref, out_ref, h_s, c_s):
    t = pl.program_id(0)
    o1 = one_ref[0, 0]

    @pl.when(t == 0)
    def _init():
        h_s[...] = h0_ref[...]
        c_s[...] = c0_ref[...]

    LT = LT_ref[...]

    CH = 4
    U0 = jnp.concatenate([xT_ref[0], h_s[...]], axis=1)
    U1c, U2c = [], []
    for g in range(B // CH):
        u0 = U0[g * CH:(g + 1) * CH].reshape(CH * CAT, N)
        u1 = jnp.dot(u0, LT, preferred_element_type=jnp.float32)
        u2 = 2.0 * jnp.dot(u1, LT, preferred_element_type=jnp.float32) - u0
        U1c.append(u1.reshape(CH, CAT, N))
        U2c.append(u2.reshape(CH, CAT, N))
    U1 = jnp.concatenate(U1c, axis=0)
    U2 = jnp.concatenate(U2c, axis=0)

    bxT = bx_ref[0]
    bhT = bh_ref[0]
    wc_i = wc_ref[0, 0]
    wc_f = wc_ref[0, 1]
    wc_o = wc_ref[0, 2]
    bg_i = bg_ref[0, :HID]
    bg_f = bg_ref[0, HID:2 * HID]
    bg_c = bg_ref[0, 2 * HID:3 * HID]
    bg_o = bg_ref[0, 3 * HID:]

    for b in range(B):
        xw = (jnp.dot(Wx_ref[0, 0], U0[b, :C_IN], preferred_element_type=jnp.float32)
              + jnp.dot(Wx_ref[0, 1], U1[b, :C_IN], preferred_element_type=jnp.float32))
        xw = (xw + jnp.dot(Wx_ref[0, 2], U2[b, :C_IN], preferred_element_type=jnp.float32)) + bxT
        hw = (jnp.dot(Wh_ref[0, 0], U0[b, C_IN:], preferred_element_type=jnp.float32)
              + jnp.dot(Wh_ref[0, 1], U1[b, C_IN:], preferred_element_type=jnp.float32))
        hw = (hw + jnp.dot(Wh_ref[0, 2], U2[b, C_IN:], preferred_element_type=jnp.float32)) + bhT
        pre = o1 * xw + o1 * hw

        c_b = c_s[b]
        Ig = jax.nn.sigmoid((pre[:HID] + wc_i * c_b) + bg_i)
        Fg = jax.nn.sigmoid((pre[HID:2 * HID] + wc_f * c_b) + bg_f)
        Tc = jnp.tanh(pre[2 * HID:3 * HID] + bg_c)
        c_new = Fg * c_b + Ig * Tc
        Og = jax.nn.sigmoid((pre[3 * HID:] + wc_o * c_new) + bg_o)
        h_new = Og * jnp.tanh(c_new)

        c_s[b] = c_new
        h_s[b] = h_new
        out_ref[b, 0] = h_new


def kernel(X, L, H, C, Wx, bx, Wh, bh, wc, bg):
    LT = L.T
    XT = jnp.transpose(X, (1, 0, 3, 2))
    WxS = jnp.transpose(Wx, (0, 2, 1, 4, 3)).reshape(T_STEPS, K, GOUT, C_IN)
    WhS = jnp.transpose(Wh, (0, 2, 1, 4, 3)).reshape(T_STEPS, K, GOUT, HID)
    bxS = bx.reshape(T_STEPS, GOUT, 1)
    bhS = bh.reshape(T_STEPS, GOUT, 1)
    bgS = bg[:, :, 0].reshape(T_STEPS, GOUT, 1)
    wcr = wc[:, :, 0].reshape(T_STEPS, 3, HID, 1)
    h0T = jnp.transpose(H[0], (0, 2, 1))
    c0T = jnp.transpose(C[0], (0, 2, 1))
    one = jnp.ones((8, 128), jnp.float32)

    outT = pl.pallas_call(
        _step_kernel,
        grid=(T_STEPS,),
        in_specs=[
            pl.BlockSpec((1, B, C_IN, N), lambda t: (t, 0, 0, 0)),
            pl.BlockSpec((N, N), lambda t: (0, 0)),
            pl.BlockSpec((1, K, GOUT, C_IN), lambda t: (t, 0, 0, 0)),
            pl.BlockSpec((1, K, GOUT, HID), lambda t: (t, 0, 0, 0)),
            pl.BlockSpec((1, GOUT, 1), lambda t: (t, 0, 0)),
            pl.BlockSpec((1, GOUT, 1), lambda t: (t, 0, 0)),
            pl.BlockSpec((1, GOUT, 1), lambda t: (t, 0, 0)),
            pl.BlockSpec((1, 3, HID, 1), lambda t: (t, 0, 0, 0)),
            pl.BlockSpec((8, 128), lambda t: (0, 0)),
            pl.BlockSpec((B, HID, N), lambda t: (0, 0, 0)),
            pl.BlockSpec((B, HID, N), lambda t: (0, 0, 0)),
        ],
        out_specs=pl.BlockSpec((B, 1, HID, N), lambda t: (0, t, 0, 0)),
        out_shape=jax.ShapeDtypeStruct((B, T_STEPS, HID, N), jnp.float32),
        scratch_shapes=[
            pltpu.VMEM((B, HID, N), jnp.float32),
            pltpu.VMEM((B, HID, N), jnp.float32),
        ],
        compiler_params=pltpu.CompilerParams(
            dimension_semantics=("arbitrary",),
        ),
    )(XT, LT, WxS, WhS, bxS, bhS, bgS, wcr, one, h0T, c0T)
    return jnp.transpose(outT, (0, 1, 3, 2))
